# Initial kernel scaffold; baseline (speedup 1.0000x reference)
import functools

import jax
import jax.numpy as jnp
from jax.experimental import pallas as pl
from jax.experimental.pallas import tpu as pltpu
from jax.experimental.pallas import tpu_sc as plsc

TEMP = 0.1
MARGIN = 1.0
EPS = 1e-8

N_NODES = 10000
D = 128
N_POS = 320000
N_NEG = 320000
N_ALL = N_POS + N_NEG
N_IDX = 2 * N_ALL

GW = 128
RBLK = 3200
N_RBLK = N_ALL // RBLK
N_POS_BLK = N_POS // RBLK


def _prep_body(emb_ref, flow_ref, nemb_ref, c_ref):
    x = emb_ref[...]
    ss = jnp.sum(x * x, axis=1, keepdims=True)
    inv = 1.0 / jnp.maximum(jnp.sqrt(ss), 1e-12)
    nemb_ref[...] = (x * inv).astype(jnp.bfloat16)
    w = flow_ref[...]
    c_ref[...] = -jnp.log(w + EPS) * (1.0 / TEMP)


def _prep(embeddings, flow_weights):
    flow2d = flow_weights.reshape(2500, 128)
    return pl.pallas_call(
        _prep_body,
        out_shape=(
            jax.ShapeDtypeStruct((N_NODES, D), jnp.bfloat16),
            jax.ShapeDtypeStruct((2500, 128), jnp.float32),
        ),
    )(embeddings, flow2d)


def _sc_gather(table_bf16, idx):
    idx2 = idx.reshape(1, N_IDX)
    mesh = plsc.VectorSubcoreMesh(core_axis_name="core", subcore_axis_name="subcore")

    @functools.partial(
        pl.kernel,
        out_type=jax.ShapeDtypeStruct((N_IDX, D), jnp.bfloat16),
        mesh=mesh,
    )
    def k(x_hbm, i_hbm, o_hbm):
        def body(i_vmem, o_vmem):
            pltpu.sync_copy(x_hbm.at[i_vmem.at[0]], o_vmem)

        pltpu.emit_pipeline(
            body,
            grid=(N_IDX // GW,),
            in_specs=[pl.BlockSpec((1, GW), lambda i: (0, i))],
            out_specs=[pl.BlockSpec((GW, D), lambda i: (i, 0))],
            core_axis_name=("core", "subcore"),
            dimension_semantics=(pltpu.PARALLEL,),
        )(i_hbm, o_hbm)

    return k(table_bf16, idx2)


def _loss_body(a_ref, b_ref, c_ref, out_ref):
    i = pl.program_id(0)
    a = a_ref[...].astype(jnp.float32)
    b = b_ref[...].astype(jnp.float32)
    sims = jnp.sum(a * b, axis=1)
    c = c_ref[...].reshape(RBLK)
    pos_contrib = jnp.sum(sims * c)
    neg_contrib = jnp.sum(jnp.maximum(sims * (1.0 / TEMP) - MARGIN, 0.0))
    contrib = jnp.where(i < N_POS_BLK, pos_contrib, neg_contrib) * (1.0 / N_ALL)

    @pl.when(i == 0)
    def _():
        out_ref[0, 0] = 0.0

    out_ref[0, 0] += contrib


def _loss(gathered, c):
    c3 = c.reshape(N_POS_BLK, 1, RBLK)
    out = pl.pallas_call(
        _loss_body,
        grid=(N_RBLK,),
        in_specs=[
            pl.BlockSpec((RBLK, D), lambda i: (i, 0)),
            pl.BlockSpec((RBLK, D), lambda i: (N_RBLK + i, 0)),
            pl.BlockSpec((1, 1, RBLK), lambda i: (jnp.minimum(i, N_POS_BLK - 1), 0, 0)),
        ],
        out_specs=pl.BlockSpec((1, 1), lambda i: (0, 0)),
        out_shape=jax.ShapeDtypeStruct((1, 1), jnp.float32),
    )(gathered, gathered, c3)
    return out[0, 0]


def kernel(embeddings, positive_pairs, flow_weights, negative_pairs):
    nemb, c = _prep(embeddings, flow_weights)
    idx = jnp.concatenate(
        [positive_pairs[0], negative_pairs[0], positive_pairs[1], negative_pairs[1]]
    ).astype(jnp.int32)
    gathered = _sc_gather(nemb, idx)
    return _loss(gathered, c.reshape(-1))


# speedup vs baseline: 3.4080x; 3.4080x over previous
import functools

import jax
import jax.numpy as jnp
from jax.experimental import pallas as pl
from jax.experimental.pallas import tpu as pltpu
from jax.experimental.pallas import tpu_sc as plsc

TEMP = 0.1
MARGIN = 1.0
EPS = 1e-8

N_NODES = 10000
D = 128
N_POS = 320000
N_NEG = 320000
N_ALL = N_POS + N_NEG
N_IDX = 2 * N_ALL

GW = 128
RBLK = 3200
N_RBLK = N_ALL // RBLK
N_POS_BLK = N_POS // RBLK


def _prep_body(emb_ref, flow_ref, nemb_ref, c_ref):
    x = emb_ref[...]
    ss = jnp.sum(x * x, axis=1, keepdims=True)
    inv = 1.0 / jnp.maximum(jnp.sqrt(ss), 1e-12)
    nemb_ref[...] = x * inv
    w = flow_ref[...]
    c_ref[...] = -jnp.log(w + EPS) * (1.0 / TEMP)


def _prep(embeddings, flow_weights):
    flow2d = flow_weights.reshape(2500, 128)
    return pl.pallas_call(
        _prep_body,
        out_shape=(
            jax.ShapeDtypeStruct((N_NODES, D), jnp.float32),
            jax.ShapeDtypeStruct((2500, 128), jnp.float32),
        ),
    )(embeddings, flow2d)


def _sc_gather(table, idx):
    idx2 = idx.reshape(1, N_IDX)
    mesh = plsc.VectorSubcoreMesh(core_axis_name="core", subcore_axis_name="subcore")

    @functools.partial(
        pl.kernel,
        out_type=jax.ShapeDtypeStruct((N_IDX, D), jnp.float32),
        mesh=mesh,
    )
    def k(x_hbm, i_hbm, o_hbm):
        def body(i_vmem, o_vmem):
            pltpu.sync_copy(x_hbm.at[i_vmem.at[0]], o_vmem)

        pltpu.emit_pipeline(
            body,
            grid=(N_IDX // GW,),
            in_specs=[pl.BlockSpec((1, GW), lambda i: (0, i))],
            out_specs=[pl.BlockSpec((GW, D), lambda i: (i, 0))],
            core_axis_name=("core", "subcore"),
            dimension_semantics=(pltpu.PARALLEL,),
        )(i_hbm, o_hbm)

    return k(table, idx2)


def _loss_body(a_ref, b_ref, c_ref, out_ref):
    i = pl.program_id(0)
    a = a_ref[...].astype(jnp.float32)
    b = b_ref[...].astype(jnp.float32)
    sims = jnp.sum(a * b, axis=1)
    c = c_ref[...].reshape(RBLK)
    pos_contrib = jnp.sum(sims * c)
    neg_contrib = jnp.sum(jnp.maximum(sims * (1.0 / TEMP) - MARGIN, 0.0))
    contrib = jnp.where(i < N_POS_BLK, pos_contrib, neg_contrib) * (1.0 / N_ALL)

    @pl.when(i == 0)
    def _():
        out_ref[...] = jnp.zeros((1, 128), jnp.float32)

    out_ref[...] += jnp.full((1, 128), contrib, jnp.float32)


def _loss(gathered, c):
    c3 = c.reshape(N_POS_BLK, 1, RBLK)
    out = pl.pallas_call(
        _loss_body,
        grid=(N_RBLK,),
        in_specs=[
            pl.BlockSpec((RBLK, D), lambda i: (i, 0)),
            pl.BlockSpec((RBLK, D), lambda i: (N_RBLK + i, 0)),
            pl.BlockSpec((1, 1, RBLK), lambda i: (jnp.minimum(i, N_POS_BLK - 1), 0, 0)),
        ],
        out_specs=pl.BlockSpec((1, 128), lambda i: (0, 0)),
        out_shape=jax.ShapeDtypeStruct((1, 128), jnp.float32),
    )(gathered, gathered, c3)
    return out[0, 0]


def kernel(embeddings, positive_pairs, flow_weights, negative_pairs):
    nemb, c = _prep(embeddings, flow_weights)
    idx = jnp.concatenate(
        [positive_pairs[0], negative_pairs[0], positive_pairs[1], negative_pairs[1]]
    ).astype(jnp.int32)
    gathered = _sc_gather(nemb, idx)
    return _loss(gathered, c.reshape(-1))


# speedup vs baseline: 3.6220x; 1.0628x over previous
import dataclasses
import functools

import jax
import jax.numpy as jnp
from jax.experimental import pallas as pl
from jax.experimental.pallas import tpu as pltpu
from jax.experimental.pallas import tpu_sc as plsc

TEMP = 0.1
MARGIN = 1.0
EPS = 1e-8

N_NODES = 10000
D = 128
N_POS = 320000
N_NEG = 320000
N_ALL = N_POS + N_NEG
NPAD = 10240
G_ROWS = NPAD * NPAD // 128

GW = 128
GT = 1024
RB = 6400
N_RBLK = N_ALL // RB
N_POS_BLK = N_POS // RB


def _prep_body(emb_ref, flow_ref, nemb_ref, c_ref):
    x = emb_ref[...]
    ss = jnp.sum(x * x, axis=1, keepdims=True)
    inv = 1.0 / jnp.maximum(jnp.sqrt(ss), 1e-12)
    nemb_ref[...] = (x * inv).astype(jnp.bfloat16)
    w = flow_ref[...]
    c_ref[...] = -jnp.log(w + EPS) * (1.0 / TEMP)


def _prep(emb_pad, flow_weights):
    flow2d = flow_weights.reshape(2500, 128)
    return pl.pallas_call(
        _prep_body,
        out_shape=(
            jax.ShapeDtypeStruct((NPAD, D), jnp.bfloat16),
            jax.ShapeDtypeStruct((2500, 128), jnp.float32),
        ),
    )(emb_pad, flow2d)


def _gram_body(a_ref, b_ref, o_ref):
    o_ref[...] = jax.lax.dot_general(
        a_ref[...], b_ref[...], (((1,), (1,)), ((), ())),
        preferred_element_type=jnp.float32,
    )


def _gram(nemb):
    n_t = NPAD // GT
    return pl.pallas_call(
        _gram_body,
        grid=(n_t, n_t),
        in_specs=[
            pl.BlockSpec((GT, D), lambda m, n: (m, 0)),
            pl.BlockSpec((GT, D), lambda m, n: (n, 0)),
        ],
        out_specs=pl.BlockSpec((GT, GT), lambda m, n: (m, n)),
        out_shape=jax.ShapeDtypeStruct((NPAD, NPAD), jnp.float32),
    )(nemb, nemb)


def _sc_gather_sims(g128, i_all, j_all):
    mesh = plsc.VectorSubcoreMesh(core_axis_name="core", subcore_axis_name="subcore")
    cp = pltpu.CompilerParams()
    if "needs_layout_passes" in pltpu.CompilerParams.__dataclass_fields__:
        cp = dataclasses.replace(cp, needs_layout_passes=False)

    @functools.partial(
        pl.kernel,
        out_type=jax.ShapeDtypeStruct((1, N_ALL), jnp.float32),
        mesh=mesh,
        compiler_params=cp,
        scratch_types=[
            pltpu.VMEM((GW,), jnp.int32),
            pltpu.VMEM((GW,), jnp.int32),
            pltpu.VMEM((GW, 128), jnp.float32),
        ],
    )
    def k(g_hbm, i_hbm, j_hbm, o_hbm, idx_v, lane_v, gwin):
        def body(i_vmem, j_vmem, o_vmem):
            for cc in range(GW // 16):
                sl = pl.ds(cc * 16, 16)
                p = i_vmem[0, sl] * NPAD + j_vmem[0, sl]
                idx_v[sl] = jax.lax.shift_right_logical(p, 7)
                lane_v[sl] = p & 127
            pltpu.sync_copy(g_hbm.at[idx_v], gwin)
            for cc in range(GW // 16):
                sl = pl.ds(cc * 16, 16)
                rows = jax.lax.iota(jnp.int32, 16) + (cc * 16)
                o_vmem[0, sl] = plsc.load_gather(gwin, [rows, lane_v[sl]])

        pltpu.emit_pipeline(
            body,
            grid=(N_ALL // GW,),
            in_specs=[
                pl.BlockSpec((1, GW), lambda w: (0, w)),
                pl.BlockSpec((1, GW), lambda w: (0, w)),
            ],
            out_specs=[pl.BlockSpec((1, GW), lambda w: (0, w))],
            core_axis_name=("core", "subcore"),
            dimension_semantics=(pltpu.PARALLEL,),
        )(i_hbm, j_hbm, o_hbm)

    return k(g128, i_all, j_all)


def _loss_body(s_ref, c_ref, out_ref):
    blk = pl.program_id(0)
    s = s_ref[...].reshape(RB // 128, 128)
    cr = c_ref[...].reshape(RB // 128, 128)
    pos_contrib = jnp.sum(s * cr)
    neg_contrib = jnp.sum(jnp.maximum(s * (1.0 / TEMP) - MARGIN, 0.0))
    contrib = jnp.where(blk < N_POS_BLK, pos_contrib, neg_contrib) * (1.0 / N_ALL)

    @pl.when(blk == 0)
    def _():
        out_ref[...] = jnp.zeros((1, 128), jnp.float32)

    out_ref[...] += jnp.full((1, 128), contrib, jnp.float32)


def _loss(sims, c):
    s2 = sims.reshape(N_RBLK, RB // 128, 128)
    c3 = c.reshape(N_POS_BLK, 1, RB)
    out = pl.pallas_call(
        _loss_body,
        grid=(N_RBLK,),
        in_specs=[
            pl.BlockSpec((1, RB // 128, 128), lambda i: (i, 0, 0)),
            pl.BlockSpec((1, 1, RB), lambda i: (jnp.minimum(i, N_POS_BLK - 1), 0, 0)),
        ],
        out_specs=pl.BlockSpec((1, 128), lambda i: (0, 0)),
        out_shape=jax.ShapeDtypeStruct((1, 128), jnp.float32),
    )(s2, c3)
    return out[0, 0]


def kernel(embeddings, positive_pairs, flow_weights, negative_pairs):
    emb_pad = jnp.pad(embeddings, ((0, NPAD - N_NODES), (0, 0)))
    nemb, c = _prep(emb_pad, flow_weights)
    g = _gram(nemb)
    g128 = g.reshape(G_ROWS, 128)
    i_all = jnp.concatenate([positive_pairs[0], negative_pairs[0]]).astype(jnp.int32)
    j_all = jnp.concatenate([positive_pairs[1], negative_pairs[1]]).astype(jnp.int32)
    sims = _sc_gather_sims(g128, i_all.reshape(1, N_ALL), j_all.reshape(1, N_ALL))
    return _loss(sims.reshape(-1), c.reshape(-1))


# speedup vs baseline: 8.9917x; 2.4825x over previous
import dataclasses
import functools

import jax
import jax.numpy as jnp
from jax.experimental import pallas as pl
from jax.experimental.pallas import tpu as pltpu
from jax.experimental.pallas import tpu_sc as plsc

TEMP = 0.1
MARGIN = 1.0
EPS = 1e-8

N_NODES = 10000
D = 128
N_POS = 320000
N_NEG = 320000
N_ALL = N_POS + N_NEG
NPAD = 10240
G_ROWS = NPAD * NPAD // 128

GW = 512
SUB = 128
GT = 1024


def _gram_body(e_ref, f_ref, o_ref, c_ref, cache):
    m = pl.program_id(0)
    n = pl.program_id(1)

    @pl.when(m == 0)
    def _():
        x = e_ref[...]
        ss = jnp.sum(x * x, axis=1, keepdims=True)
        inv = 1.0 / jnp.maximum(jnp.sqrt(ss), 1e-12)
        cache[pl.ds(n * GT, GT), :] = (x * inv).astype(jnp.bfloat16)

    @pl.when((m == 0) & (n == 0))
    def _():
        c_ref[...] = -jnp.log(f_ref[...] + EPS) * (1.0 / TEMP)

    a = cache[pl.ds(m * GT, GT), :]
    b = cache[pl.ds(n * GT, GT), :]
    g = jax.lax.dot_general(
        a, b, (((1,), (1,)), ((), ())), preferred_element_type=jnp.float32
    )
    o_ref[...] = g.reshape(GT, GT // 128, 128)


def _gram(emb_pad, flow2d):
    n_t = NPAD // GT
    return pl.pallas_call(
        _gram_body,
        grid=(n_t, n_t),
        in_specs=[
            pl.BlockSpec((GT, D), lambda m, n: (n, 0)),
            pl.BlockSpec((1, N_POS), lambda m, n: (0, 0)),
        ],
        out_specs=[
            pl.BlockSpec((GT, GT // 128, 128), lambda m, n: (m, n, 0)),
            pl.BlockSpec((1, N_POS), lambda m, n: (0, 0)),
        ],
        out_shape=[
            jax.ShapeDtypeStruct((NPAD, NPAD // 128, 128), jnp.float32),
            jax.ShapeDtypeStruct((1, N_POS), jnp.float32),
        ],
        scratch_shapes=[pltpu.VMEM((NPAD, D), jnp.bfloat16)],
    )(emb_pad, flow2d)


def _sc_loss(g128, i_pos, j_pos, i_neg, j_neg, c):
    mesh = plsc.VectorSubcoreMesh(core_axis_name="core", subcore_axis_name="subcore")
    cp = pltpu.CompilerParams()
    if "needs_layout_passes" in pltpu.CompilerParams.__dataclass_fields__:
        cp = dataclasses.replace(cp, needs_layout_passes=False)

    @functools.partial(
        pl.kernel,
        out_type=jax.ShapeDtypeStruct((32, 16), jnp.float32),
        mesh=mesh,
        compiler_params=cp,
        scratch_types=[
            pltpu.VMEM((GW // SUB, SUB), jnp.int32),
            pltpu.VMEM((GW,), jnp.int32),
            pltpu.VMEM((GW, 128), jnp.float32),
            pltpu.VMEM((16,), jnp.float32),
            pltpu.SemaphoreType.DMA,
        ],
    )
    def k(g_hbm, ip_hbm, jp_hbm, in_hbm, jn_hbm, c_hbm, o_hbm,
          idx_v, lane_v, gwin, acc, sem):
        acc[...] = jnp.zeros((16,), jnp.float32)

        def gather_window(i_vmem, j_vmem):
            for s in range(GW // SUB):
                for cc in range(SUB // 16):
                    o = s * SUB + cc * 16
                    sl = pl.ds(o, 16)
                    p = i_vmem[0, sl] * NPAD + j_vmem[0, sl]
                    idx_v[s, pl.ds(cc * 16, 16)] = jax.lax.shift_right_logical(p, 7)
                    lane_v[sl] = p & 127
            cps = [
                pltpu.async_copy(
                    g_hbm.at[idx_v.at[s]], gwin.at[pl.ds(s * SUB, SUB)], sem
                )
                for s in range(GW // SUB)
            ]
            for c_ in cps:
                c_.wait()

        def pos_body(i_vmem, j_vmem, c_vmem):
            gather_window(i_vmem, j_vmem)
            for cc in range(GW // 16):
                sl = pl.ds(cc * 16, 16)
                rows = jax.lax.iota(jnp.int32, 16) + (cc * 16)
                sims = plsc.load_gather(gwin, [rows, lane_v[sl]])
                acc[...] = acc[...] + sims * c_vmem[0, sl]

        def neg_body(i_vmem, j_vmem):
            gather_window(i_vmem, j_vmem)
            for cc in range(GW // 16):
                sl = pl.ds(cc * 16, 16)
                rows = jax.lax.iota(jnp.int32, 16) + (cc * 16)
                sims = plsc.load_gather(gwin, [rows, lane_v[sl]])
                acc[...] = acc[...] + jnp.maximum(
                    sims * (1.0 / TEMP) - MARGIN, 0.0
                )

        pairspec = pl.BlockSpec((1, GW), lambda w: (0, w))
        pltpu.emit_pipeline(
            pos_body,
            grid=(N_POS // GW,),
            in_specs=[pairspec, pairspec, pairspec],
            out_specs=[],
            core_axis_name=("core", "subcore"),
            dimension_semantics=(pltpu.PARALLEL,),
        )(ip_hbm, jp_hbm, c_hbm)
        pltpu.emit_pipeline(
            neg_body,
            grid=(N_NEG // GW,),
            in_specs=[pairspec, pairspec],
            out_specs=[],
            core_axis_name=("core", "subcore"),
            dimension_semantics=(pltpu.PARALLEL,),
        )(in_hbm, jn_hbm)

        wid = jax.lax.axis_index("core") * 16 + jax.lax.axis_index("subcore")
        pltpu.sync_copy(acc, o_hbm.at[wid])

    return k(g128, i_pos, j_pos, i_neg, j_neg, c)


def kernel(embeddings, positive_pairs, flow_weights, negative_pairs):
    emb_pad = jnp.pad(embeddings, ((0, NPAD - N_NODES), (0, 0)))
    g, c = _gram(emb_pad, flow_weights.reshape(1, N_POS))
    g128 = g.reshape(G_ROWS, 128)
    partials = _sc_loss(
        g128,
        positive_pairs[0].astype(jnp.int32).reshape(1, N_POS),
        positive_pairs[1].astype(jnp.int32).reshape(1, N_POS),
        negative_pairs[0].astype(jnp.int32).reshape(1, N_NEG),
        negative_pairs[1].astype(jnp.int32).reshape(1, N_NEG),
        c,
    )
    return jnp.sum(partials) * (1.0 / N_ALL)


# speedup vs baseline: 10.3606x; 1.1522x over previous
import dataclasses
import functools

import jax
import jax.numpy as jnp
from jax.experimental import pallas as pl
from jax.experimental.pallas import tpu as pltpu
from jax.experimental.pallas import tpu_sc as plsc

TEMP = 0.1
MARGIN = 1.0
EPS = 1e-8

N_NODES = 10000
D = 128
N_POS = 320000
N_NEG = 320000
N_ALL = N_POS + N_NEG
NPAD = 10240
G_ROWS = NPAD // 2 * NPAD // 128

GW = 640
SUB = 128
GT = 1024


def _gram_body(e_ref, f_ref, o_ref, c_ref, cache):
    m = pl.program_id(0)
    n = pl.program_id(1)

    @pl.when(m == 0)
    def _():
        x = e_ref[...]
        ss = jnp.sum(x * x, axis=1, keepdims=True)
        inv = 1.0 / jnp.maximum(jnp.sqrt(ss), 1e-12)
        cache[pl.ds(n * GT, GT), :] = (x * inv).astype(jnp.bfloat16)

    @pl.when((m == 0) & (n == 0))
    def _():
        c_ref[...] = -jnp.log(f_ref[...] + EPS) * (1.0 / TEMP)

    a = cache[pl.ds(m * GT, GT), :]
    b = cache[pl.ds(n * GT, GT), :]
    g = jax.lax.dot_general(
        a, b, (((1,), (1,)), ((), ())), preferred_element_type=jnp.float32
    )
    gp = pltpu.bitcast(g.astype(jnp.bfloat16), jnp.int32)
    o_ref[...] = gp.reshape(GT // 2, GT // 128, 128)


def _gram(emb_pad, flow2d):
    n_t = NPAD // GT
    return pl.pallas_call(
        _gram_body,
        grid=(n_t, n_t),
        in_specs=[
            pl.BlockSpec((GT, D), lambda m, n: (n, 0)),
            pl.BlockSpec((1, N_POS), lambda m, n: (0, 0)),
        ],
        out_specs=[
            pl.BlockSpec((GT // 2, GT // 128, 128), lambda m, n: (m, n, 0)),
            pl.BlockSpec((1, N_POS), lambda m, n: (0, 0)),
        ],
        out_shape=[
            jax.ShapeDtypeStruct((NPAD // 2, NPAD // 128, 128), jnp.int32),
            jax.ShapeDtypeStruct((1, N_POS), jnp.float32),
        ],
        scratch_shapes=[pltpu.VMEM((NPAD, D), jnp.bfloat16)],
    )(emb_pad, flow2d)


def _sc_loss(g128, i_pos, j_pos, i_neg, j_neg, c):
    mesh = plsc.VectorSubcoreMesh(core_axis_name="core", subcore_axis_name="subcore")
    cp = pltpu.CompilerParams()
    if "needs_layout_passes" in pltpu.CompilerParams.__dataclass_fields__:
        cp = dataclasses.replace(cp, needs_layout_passes=False)

    @functools.partial(
        pl.kernel,
        out_type=jax.ShapeDtypeStruct((32, 16), jnp.float32),
        mesh=mesh,
        compiler_params=cp,
        scratch_types=[
            pltpu.VMEM((GW // SUB, SUB), jnp.int32),
            pltpu.VMEM((GW,), jnp.int32),
            pltpu.VMEM((GW,), jnp.int32),
            pltpu.VMEM((GW, 128), jnp.int32),
            pltpu.VMEM((16,), jnp.float32),
            pltpu.SemaphoreType.DMA,
        ],
    )
    def k(g_hbm, ip_hbm, jp_hbm, in_hbm, jn_hbm, c_hbm, o_hbm,
          idx_v, lane_v, half_v, gwin, acc, sem):
        acc[...] = jnp.zeros((16,), jnp.float32)

        def gather_window(i_vmem, j_vmem):
            for s in range(GW // SUB):
                for cc in range(SUB // 16):
                    o = s * SUB + cc * 16
                    sl = pl.ds(o, 16)
                    iv = i_vmem[0, sl]
                    p = jax.lax.shift_right_logical(iv, 1) * NPAD + j_vmem[0, sl]
                    idx_v[s, pl.ds(cc * 16, 16)] = jax.lax.shift_right_logical(p, 7)
                    lane_v[sl] = p & 127
                    half_v[sl] = iv & 1
            cps = [
                pltpu.async_copy(
                    g_hbm.at[idx_v.at[s]], gwin.at[pl.ds(s * SUB, SUB)], sem
                )
                for s in range(GW // SUB)
            ]
            for c_ in cps:
                c_.wait()

        def select_sims(cc):
            sl = pl.ds(cc * 16, 16)
            rows = jax.lax.iota(jnp.int32, 16) + (cc * 16)
            w = plsc.load_gather(gwin, [rows, lane_v[sl]])
            bits = jnp.where(
                half_v[sl] == 0,
                jax.lax.shift_left(w, 16),
                w & jnp.int32(-65536),
            )
            return plsc.bitcast(bits, jnp.float32), sl

        def pos_body(i_vmem, j_vmem, c_vmem):
            gather_window(i_vmem, j_vmem)
            for cc in range(GW // 16):
                sims, sl = select_sims(cc)
                acc[...] = acc[...] + sims * c_vmem[0, sl]

        def neg_body(i_vmem, j_vmem):
            gather_window(i_vmem, j_vmem)
            for cc in range(GW // 16):
                sims, _ = select_sims(cc)
                acc[...] = acc[...] + jnp.maximum(
                    sims * (1.0 / TEMP) - MARGIN, 0.0
                )

        pairspec = pl.BlockSpec((1, GW), lambda w: (0, w))
        pltpu.emit_pipeline(
            pos_body,
            grid=(N_POS // GW,),
            in_specs=[pairspec, pairspec, pairspec],
            out_specs=[],
            core_axis_name=("core", "subcore"),
            dimension_semantics=(pltpu.PARALLEL,),
        )(ip_hbm, jp_hbm, c_hbm)
        pltpu.emit_pipeline(
            neg_body,
            grid=(N_NEG // GW,),
            in_specs=[pairspec, pairspec],
            out_specs=[],
            core_axis_name=("core", "subcore"),
            dimension_semantics=(pltpu.PARALLEL,),
        )(in_hbm, jn_hbm)

        wid = jax.lax.axis_index("core") * 16 + jax.lax.axis_index("subcore")
        pltpu.sync_copy(acc, o_hbm.at[wid])

    return k(g128, i_pos, j_pos, i_neg, j_neg, c)


def kernel(embeddings, positive_pairs, flow_weights, negative_pairs):
    emb_pad = jnp.pad(embeddings, ((0, NPAD - N_NODES), (0, 0)))
    g, c = _gram(emb_pad, flow_weights.reshape(1, N_POS))
    g128 = g.reshape(G_ROWS, 128)
    partials = _sc_loss(
        g128,
        positive_pairs[0].astype(jnp.int32).reshape(1, N_POS),
        positive_pairs[1].astype(jnp.int32).reshape(1, N_POS),
        negative_pairs[0].astype(jnp.int32).reshape(1, N_NEG),
        negative_pairs[1].astype(jnp.int32).reshape(1, N_NEG),
        c,
    )
    return jnp.sum(partials) * (1.0 / N_ALL)
